# baseline (speedup 1.0000x reference)
import functools
import math

import jax
import jax.numpy as jnp
import numpy as np
from jax import lax
from jax.experimental import pallas as pl
from jax.experimental.pallas import tpu as pltpu
from jax.experimental.pallas import tpu_sc as plsc

B, T, C = 32, 384, 256
N_HEAD, N_KV_HEAD = 4, 2
HEAD_DIM = C // N_HEAD
HALF = HEAD_DIM // 2
E = 8
HID = int(8 / 3 * C)
HPAD = 768
N = B * T
TILE = 256
NTOK_TILES = N // TILE
PBUF = N * 2 + E * TILE
NTILES = PBUF // TILE
NW = 32
TPW = N // NW


def _rope_tables():
    inv_freq = 1.0 / (10000.0 ** (np.arange(0, HEAD_DIM, 2).astype(np.float32) / HEAD_DIM))
    t = np.arange(T).astype(np.float32)
    freqs = np.einsum('i,j->ij', t, inv_freq)
    emb = np.concatenate((freqs, freqs), axis=-1)
    return jnp.asarray(np.cos(emb)), jnp.asarray(np.sin(emb))


QB = 128


def _ar_body(x_ref, w1_ref, qwt_ref, kwt_ref, vwt_ref, owt_ref, cos_ref,
             sin_ref, w2_ref, rwt_ref, tril_ref, tril8_ref,
             x1_ref, x2_ref, df_ref, texp_ref, run_ref):
    b = pl.program_id(0)

    @pl.when(b == 0)
    def _():
        run_ref[...] = jnp.zeros((1, E), jnp.float32)

    @pl.when(b < B)
    def _():
        xb = x_ref[0]
        xn = xb * lax.rsqrt(jnp.mean(xb * xb, axis=-1, keepdims=True) + 1e-6)
        xn = (xn * w1_ref[...]).astype(jnp.bfloat16)
        q = jnp.dot(xn, qwt_ref[...], preferred_element_type=jnp.float32)
        k = jnp.dot(xn, kwt_ref[...], preferred_element_type=jnp.float32)
        v = jnp.dot(xn, vwt_ref[...], preferred_element_type=jnp.float32)
        cos = cos_ref[...]
        sin = sin_ref[...]

        def rope(u):
            ur = jnp.concatenate([-u[:, HALF:], u[:, :HALF]], axis=1)
            return u * cos + ur * sin

        scale = 1.0 / math.sqrt(HEAD_DIM)
        ys = []
        kh_c = [rope(k[:, h * HEAD_DIM:(h + 1) * HEAD_DIM]).astype(jnp.bfloat16)
                for h in range(N_KV_HEAD)]
        for h in range(N_HEAD):
            qh = rope(q[:, h * HEAD_DIM:(h + 1) * HEAD_DIM]).astype(jnp.bfloat16)
            kh = kh_c[h // 2]
            vh = v[:, (h // 2) * HEAD_DIM:(h // 2 + 1) * HEAD_DIM].astype(jnp.bfloat16)
            blocks = []
            for qb in range(T // QB):
                kl = (qb + 1) * QB
                qhb = qh[qb * QB:(qb + 1) * QB]
                s = lax.dot_general(qhb, kh[:kl], (((1,), (1,)), ((), ())),
                                    preferred_element_type=jnp.float32) * scale
                rows = qb * QB + lax.broadcasted_iota(jnp.int32, (QB, kl), 0)
                cols = lax.broadcasted_iota(jnp.int32, (QB, kl), 1)
                s = jnp.where(rows >= cols, s, -1e30)
                m = jnp.max(s, axis=1, keepdims=True)
                p = jnp.exp(s - m)
                a = (p / jnp.sum(p, axis=1, keepdims=True)).astype(jnp.bfloat16)
                blocks.append(jnp.dot(a, vh[:kl], preferred_element_type=jnp.float32))
            ys.append(jnp.concatenate(blocks, axis=0))
        y = jnp.concatenate(ys, axis=1).astype(jnp.bfloat16)
        x1 = jnp.dot(y, owt_ref[...], preferred_element_type=jnp.float32) + xb
        x1_ref[0] = x1

        x2 = x1 * lax.rsqrt(jnp.mean(x1 * x1, axis=-1, keepdims=True) + 1e-6)
        x2 = x2 * w2_ref[...]
        x2_ref[...] = x2.astype(jnp.bfloat16)

        logits = jnp.dot(x2, rwt_ref[...], preferred_element_type=jnp.float32)
        lm = jnp.max(logits, axis=1, keepdims=True)
        ex = jnp.exp(logits - lm)
        probs = ex / jnp.sum(ex, axis=1, keepdims=True)

        lane = lax.broadcasted_iota(jnp.int32, (T, E), 1)
        p0 = jnp.max(probs, axis=1, keepdims=True)
        i0 = jnp.min(jnp.where(probs == p0, lane, E), axis=1, keepdims=True)
        masked = jnp.where(lane == i0, -1.0, probs)
        p1 = jnp.max(masked, axis=1, keepdims=True)
        i1 = jnp.min(jnp.where(masked == p1, lane, E), axis=1, keepdims=True)
        ssum = p0 + p1 + 1e-9
        g0 = p0 / ssum
        g1 = p1 / ssum

        oh0 = (lane == i0).astype(jnp.float32)
        oh1 = (lane == i1).astype(jnp.float32)
        oh = oh0 + oh1
        prefix = jnp.dot(tril_ref[...], oh, preferred_element_type=jnp.float32)
        run = run_ref[...]
        r0 = jnp.sum((prefix + run) * oh0, axis=1, keepdims=True)
        r1 = jnp.sum((prefix + run) * oh1, axis=1, keepdims=True)
        run_ref[...] = run + jnp.sum(oh, axis=0, keepdims=True)

        df_ref[b] = jnp.concatenate(
            [i0.astype(jnp.float32), i1.astype(jnp.float32), r0, r1, g0, g1,
             jnp.zeros((T, 2), jnp.float32)], axis=1)

    @pl.when(b == B)
    def _():
        cnt = run_ref[...]
        padded = jnp.ceil(cnt / TILE) * TILE
        offs = jnp.dot(padded, tril8_ref[...], preferred_element_type=jnp.float32)
        ends = (offs + padded) / TILE
        tt = lax.broadcasted_iota(jnp.int32, (1, 128), 1).astype(jnp.float32)
        acc = jnp.zeros((1, 128), jnp.float32)
        for e in range(E):
            acc = acc + (tt >= ends[0, e]).astype(jnp.float32)
        texp_ref[...] = jnp.minimum(acc, float(E - 1))

        mfall = df_ref[...]
        d0 = mfall[:, :, 2]
        d1 = mfall[:, :, 3]
        for e in range(E):
            d0 = d0 + jnp.where(mfall[:, :, 0] == float(e), offs[0, e], 0.0)
            d1 = d1 + jnp.where(mfall[:, :, 1] == float(e), offs[0, e], 0.0)
        df_ref[...] = jnp.concatenate(
            [d0[:, :, None], d1[:, :, None], mfall[:, :, 2:]], axis=2)


def _attn_router(x, ln1_w, qW, kW, vW, oW, ln2_w, routerW):
    cos, sin = _rope_tables()
    tril = np.tril(np.ones((T, T), np.float32), k=-1)
    tril8 = np.triu(np.ones((E, E), np.float32), k=1)
    last = B - 1
    return pl.pallas_call(
        _ar_body,
        grid=(B + 1,),
        in_specs=[
            pl.BlockSpec((1, T, C), lambda b: (jnp.minimum(b, last), 0, 0)),
            pl.BlockSpec((1, C), lambda b: (0, 0)),
            pl.BlockSpec((C, C), lambda b: (0, 0)),
            pl.BlockSpec((C, N_KV_HEAD * HEAD_DIM), lambda b: (0, 0)),
            pl.BlockSpec((C, N_KV_HEAD * HEAD_DIM), lambda b: (0, 0)),
            pl.BlockSpec((C, C), lambda b: (0, 0)),
            pl.BlockSpec((T, HEAD_DIM), lambda b: (0, 0)),
            pl.BlockSpec((T, HEAD_DIM), lambda b: (0, 0)),
            pl.BlockSpec((1, C), lambda b: (0, 0)),
            pl.BlockSpec((C, E), lambda b: (0, 0)),
            pl.BlockSpec((T, T), lambda b: (0, 0)),
            pl.BlockSpec((E, E), lambda b: (0, 0)),
        ],
        out_specs=[
            pl.BlockSpec((1, T, C), lambda b: (jnp.minimum(b, last), 0, 0)),
            pl.BlockSpec((T, C), lambda b: (jnp.minimum(b, last), 0)),
            pl.BlockSpec((B, T, E), lambda b: (0, 0, 0)),
            pl.BlockSpec((1, 128), lambda b: (0, 0)),
        ],
        out_shape=[
            jax.ShapeDtypeStruct((B, T, C), jnp.float32),
            jax.ShapeDtypeStruct((N, C), jnp.bfloat16),
            jax.ShapeDtypeStruct((B, T, E), jnp.float32),
            jax.ShapeDtypeStruct((1, 128), jnp.float32),
        ],
        scratch_shapes=[pltpu.VMEM((1, E), jnp.float32)],
    )(x, ln1_w.reshape(1, C), qW.T.astype(jnp.bfloat16),
      kW.T.astype(jnp.bfloat16), vW.T.astype(jnp.bfloat16),
      oW.T.astype(jnp.bfloat16), cos, sin, ln2_w.reshape(1, C), routerW.T,
      jnp.asarray(tril), jnp.asarray(tril8))


CH = 128


def _dispatch_body(x2_hbm, d0_hbm, d1_hbm, buf_hbm, d_v0, d_v1, rows_v, sem):
    wid = lax.axis_index("s") * 2 + lax.axis_index("c")

    def chunk(ci):
        base = wid * TPW + ci * CH
        pltpu.sync_copy(x2_hbm.at[pl.ds(base, CH)], rows_v)
        pltpu.sync_copy(d0_hbm.at[pl.ds(base, CH)], d_v0)
        pltpu.sync_copy(d1_hbm.at[pl.ds(base, CH)], d_v1)
        c0 = pltpu.async_copy(rows_v, buf_hbm.at[d_v0], sem)
        c1 = pltpu.async_copy(rows_v, buf_hbm.at[d_v1], sem)
        c0.wait()
        c1.wait()

    pl.loop(0, TPW // CH)(chunk)


def _dispatch(x2i, d0, d1):
    mesh = plsc.VectorSubcoreMesh(core_axis_name="c", subcore_axis_name="s")
    f = pl.kernel(
        _dispatch_body,
        mesh=mesh,
        out_type=jax.ShapeDtypeStruct((PBUF, C // 2), jnp.int32),
        scratch_types=[
            pltpu.VMEM((CH,), jnp.int32),
            pltpu.VMEM((CH,), jnp.int32),
            pltpu.VMEM((CH, C // 2), jnp.int32),
            pltpu.SemaphoreType.DMA,
        ],
    )
    return f(x2i, d0, d1)


def _gmm_body(texp_ref, xb_ref, w1_ref, w3_ref, w2_ref, y_ref):
    xb = xb_ref[...]
    h1 = lax.dot_general(xb, w1_ref[0], (((1,), (1,)), ((), ())),
                         preferred_element_type=jnp.float32)
    h3 = lax.dot_general(xb, w3_ref[0], (((1,), (1,)), ((), ())),
                         preferred_element_type=jnp.float32)
    h = (h1 * (1.0 / (1.0 + jnp.exp(-h1))) * h3).astype(jnp.bfloat16)
    y = lax.dot_general(h, w2_ref[0], (((1,), (1,)), ((), ())),
                        preferred_element_type=jnp.float32)
    y_ref[...] = y.astype(jnp.bfloat16)


def _grouped_mlp(texp, buf, W1p, W3p, W2p):
    grid_spec = pltpu.PrefetchScalarGridSpec(
        num_scalar_prefetch=1,
        grid=(NTILES,),
        in_specs=[
            pl.BlockSpec((TILE, C), lambda t, s: (t, 0)),
            pl.BlockSpec((1, HPAD, C), lambda t, s: (s[t], 0, 0)),
            pl.BlockSpec((1, HPAD, C), lambda t, s: (s[t], 0, 0)),
            pl.BlockSpec((1, C, HPAD), lambda t, s: (s[t], 0, 0)),
        ],
        out_specs=pl.BlockSpec((TILE, C), lambda t, s: (t, 0)),
    )
    return pl.pallas_call(
        _gmm_body,
        grid_spec=grid_spec,
        out_shape=jax.ShapeDtypeStruct((PBUF, C), jnp.bfloat16),
    )(texp, buf, W1p, W3p, W2p)


CCH = 64


def _gather_body(y_hbm, d0_hbm, d1_hbm, r0_hbm, r1_hbm,
                 d_v0, d_v1, ry0_v, ry1_v, sem):
    wid = lax.axis_index("s") * 2 + lax.axis_index("c")

    def chunk(ci):
        base = wid * TPW + ci * CCH
        pltpu.sync_copy(d0_hbm.at[pl.ds(base, CCH)], d_v0)
        pltpu.sync_copy(d1_hbm.at[pl.ds(base, CCH)], d_v1)
        c0 = pltpu.async_copy(y_hbm.at[d_v0], ry0_v, sem)
        c1 = pltpu.async_copy(y_hbm.at[d_v1], ry1_v, sem)
        c0.wait()
        c1.wait()
        pltpu.sync_copy(ry0_v, r0_hbm.at[pl.ds(base, CCH)])
        pltpu.sync_copy(ry1_v, r1_hbm.at[pl.ds(base, CCH)])

    pl.loop(0, TPW // CCH)(chunk)


def _gather2(yi, d0, d1):
    mesh = plsc.VectorSubcoreMesh(core_axis_name="c", subcore_axis_name="s")
    f = pl.kernel(
        _gather_body,
        mesh=mesh,
        out_type=[
            jax.ShapeDtypeStruct((N, C // 2), jnp.int32),
            jax.ShapeDtypeStruct((N, C // 2), jnp.int32),
        ],
        scratch_types=[
            pltpu.VMEM((CCH,), jnp.int32),
            pltpu.VMEM((CCH,), jnp.int32),
            pltpu.VMEM((CCH, C // 2), jnp.int32),
            pltpu.VMEM((CCH, C // 2), jnp.int32),
            pltpu.SemaphoreType.DMA,
        ],
    )
    return f(yi, d0, d1)


def _epilogue_body(x1_ref, ry0_ref, ry1_ref, g0_ref, g1_ref, out_ref):
    out_ref[...] = (x1_ref[...]
                    + g0_ref[...] * ry0_ref[...].astype(jnp.float32)
                    + g1_ref[...] * ry1_ref[...].astype(jnp.float32))


def _epilogue(x1f, ry0, ry1, g0, g1):
    return pl.pallas_call(
        _epilogue_body,
        grid=(NTOK_TILES,),
        in_specs=[
            pl.BlockSpec((TILE, C), lambda t: (t, 0)),
            pl.BlockSpec((TILE, C), lambda t: (t, 0)),
            pl.BlockSpec((TILE, C), lambda t: (t, 0)),
            pl.BlockSpec((TILE, 1), lambda t: (t, 0)),
            pl.BlockSpec((TILE, 1), lambda t: (t, 0)),
        ],
        out_specs=pl.BlockSpec((TILE, C), lambda t: (t, 0)),
        out_shape=jax.ShapeDtypeStruct((N, C), jnp.float32),
    )(x1f, ry0, ry1, g0, g1)


def kernel(x, ln1_w, qW, kW, vW, oW, ln2_w, routerW, W1, W2, W3):
    x1, x2, df, texpf = _attn_router(x, ln1_w, qW, kW, vW, oW, ln2_w, routerW)
    x1f = x1.reshape(N, C)
    texp = texpf[0, :NTILES].astype(jnp.int32)

    d0 = df[:, :, 0].reshape(N).astype(jnp.int32)
    d1 = df[:, :, 1].reshape(N).astype(jnp.int32)
    g0 = df[:, :, 4].reshape(N, 1)
    g1 = df[:, :, 5].reshape(N, 1)

    x2i = lax.bitcast_convert_type(x2.reshape(N, C // 2, 2), jnp.int32)
    bufi = _dispatch(x2i, d0, d1)
    buf = lax.bitcast_convert_type(bufi, jnp.bfloat16).reshape(PBUF, C)

    W1p = jnp.pad(W1.astype(jnp.bfloat16), ((0, 0), (0, HPAD - HID), (0, 0)))
    W3p = jnp.pad(W3.astype(jnp.bfloat16), ((0, 0), (0, HPAD - HID), (0, 0)))
    W2p = jnp.pad(W2.astype(jnp.bfloat16), ((0, 0), (0, 0), (0, HPAD - HID)))
    y = _grouped_mlp(texp, buf, W1p, W3p, W2p)

    yi = lax.bitcast_convert_type(y.reshape(PBUF, C // 2, 2), jnp.int32)
    ry0i, ry1i = _gather2(yi, d0, d1)
    ry0 = lax.bitcast_convert_type(ry0i, jnp.bfloat16).reshape(N, C)
    ry1 = lax.bitcast_convert_type(ry1i, jnp.bfloat16).reshape(N, C)
    out = _epilogue(x1f, ry0, ry1, g0, g1)
    return out.reshape(B, T, C)
